# Initial kernel scaffold; baseline (speedup 1.0000x reference)
import jax
import jax.numpy as jnp
from jax.experimental import pallas as pl


def kernel(x, edge_index, Wl1, Wr1, att1, b1, Wl2, Wr2, att2, b2, Wlin, blin):
    raise NotImplementedError("write your pallas kernel here")


# speedup vs baseline: 7.9293x; 7.9293x over previous
import functools

import jax
import jax.numpy as jnp
from jax import lax
from jax.experimental import pallas as pl
from jax.experimental.pallas import tpu as pltpu
from jax.experimental.pallas import tpu_sc as plsc

N = 10000
NP = 10240
E = 320000
CH = 64
NSUB = 16
NCORE = 2
NWORK = NSUB * NCORE
CPT = 157
EP = NWORK * CPT * CH
ROWS_PER_TILE = NP // NSUB


def _tc_precompute(x, Wl, Wr, att, blk=256):
    n, d_in = x.shape
    d = Wl.shape[1]
    g = n // blk

    def body(x_ref, wl_ref, wr_ref, att_ref, xl_ref, xr_ref, c_ref):
        xb = x_ref[...]
        xl = jnp.dot(xb, wl_ref[...], preferred_element_type=jnp.float32,
                     precision=lax.Precision.HIGHEST)
        xr = jnp.dot(xb, wr_ref[...], preferred_element_type=jnp.float32,
                     precision=lax.Precision.HIGHEST)
        xl_ref[...] = xl
        xr_ref[...] = xr
        z = xl + xr
        lr = jnp.maximum(z, 0.2 * z)
        c = jnp.sum(lr * att_ref[...], axis=1)
        c_ref[...] = c[None, None, :]

    return pl.pallas_call(
        body,
        grid=(g,),
        in_specs=[
            pl.BlockSpec((blk, d_in), lambda i: (i, 0)),
            pl.BlockSpec((d_in, d), lambda i: (0, 0)),
            pl.BlockSpec((d_in, d), lambda i: (0, 0)),
            pl.BlockSpec((1, d), lambda i: (0, 0)),
        ],
        out_specs=[
            pl.BlockSpec((blk, d), lambda i: (i, 0)),
            pl.BlockSpec((blk, d), lambda i: (i, 0)),
            pl.BlockSpec((1, 1, blk), lambda i: (i, 0, 0)),
        ],
        out_shape=[
            jax.ShapeDtypeStruct((n, d), jnp.float32),
            jax.ShapeDtypeStruct((n, d), jnp.float32),
            jax.ShapeDtypeStruct((g, 1, blk), jnp.float32),
        ],
    )(x, Wl, Wr, att.reshape(1, -1))


def _tc_combine_next(acc0, acc1, xl, b, Wl2, Wr2, att2, blk=256):
    n, d = xl.shape
    d2 = Wl2.shape[1]
    g = n // blk
    w = acc0.shape[1]

    def body(a0_ref, a1_ref, xl_ref, b_ref, wl_ref, wr_ref, att_ref,
             xl2_ref, xr2_ref, c2_ref):
        a0 = a0_ref[...]
        a1 = a1_ref[...]
        numer = a0[:, :d] + a1[:, :d] + xl_ref[...]
        denom = a0[:, d:d + 1] + a1[:, d:d + 1] + 1.0
        h = numer / (denom + 1e-16) + b_ref[...]
        h = jnp.maximum(h, 0.01 * h)
        xl2 = jnp.dot(h, wl_ref[...], preferred_element_type=jnp.float32,
                      precision=lax.Precision.HIGHEST)
        xr2 = jnp.dot(h, wr_ref[...], preferred_element_type=jnp.float32,
                      precision=lax.Precision.HIGHEST)
        xl2_ref[...] = xl2
        xr2_ref[...] = xr2
        z = xl2 + xr2
        lr = jnp.maximum(z, 0.2 * z)
        c2 = jnp.sum(lr * att_ref[...], axis=1)
        c2_ref[...] = c2[None, None, :]

    return pl.pallas_call(
        body,
        grid=(g,),
        in_specs=[
            pl.BlockSpec((blk, w), lambda i: (i, 0)),
            pl.BlockSpec((blk, w), lambda i: (i, 0)),
            pl.BlockSpec((blk, d), lambda i: (i, 0)),
            pl.BlockSpec((1, d), lambda i: (0, 0)),
            pl.BlockSpec((d, d2), lambda i: (0, 0)),
            pl.BlockSpec((d, d2), lambda i: (0, 0)),
            pl.BlockSpec((1, d2), lambda i: (0, 0)),
        ],
        out_specs=[
            pl.BlockSpec((blk, d2), lambda i: (i, 0)),
            pl.BlockSpec((blk, d2), lambda i: (i, 0)),
            pl.BlockSpec((1, 1, blk), lambda i: (i, 0, 0)),
        ],
        out_shape=[
            jax.ShapeDtypeStruct((n, d2), jnp.float32),
            jax.ShapeDtypeStruct((n, d2), jnp.float32),
            jax.ShapeDtypeStruct((g, 1, blk), jnp.float32),
        ],
    )(acc0, acc1, xl, b.reshape(1, -1), Wl2, Wr2, att2.reshape(1, -1))


def _tc_combine_head(acc0, acc1, xl, b, wlin, blin, blk=256):
    n, d = xl.shape
    g = n // blk
    w = acc0.shape[1]

    def body(a0_ref, a1_ref, xl_ref, b_ref, wlin_ref, blin_ref, y_ref):
        a0 = a0_ref[...]
        a1 = a1_ref[...]
        dd = w - 16
        numer = a0[:, :d] + a1[:, :d] + xl_ref[:, :d]
        denom = a0[:, dd:dd + 1] + a1[:, dd:dd + 1] + 1.0
        h = numer / (denom + 1e-16) + b_ref[...]
        y = jnp.sum(h * wlin_ref[...], axis=1) + blin_ref[0, 0]
        y_ref[...] = y[None, None, :]

    return pl.pallas_call(
        body,
        grid=(g,),
        in_specs=[
            pl.BlockSpec((blk, w), lambda i: (i, 0)),
            pl.BlockSpec((blk, w), lambda i: (i, 0)),
            pl.BlockSpec((blk, d), lambda i: (i, 0)),
            pl.BlockSpec((1, d), lambda i: (0, 0)),
            pl.BlockSpec((1, d), lambda i: (0, 0)),
            pl.BlockSpec((1, 1), lambda i: (0, 0)),
        ],
        out_specs=pl.BlockSpec((1, 1, blk), lambda i: (i, 0, 0)),
        out_shape=jax.ShapeDtypeStruct((g, 1, blk), jnp.float32),
    )(acc0, acc1, xl, b.reshape(1, -1), wlin.reshape(1, -1),
      blin.reshape(1, 1))


def _make_sc_edge_pass(d):
    wrow = d + 16
    nslc = d // 16

    @functools.partial(
        pl.kernel,
        mesh=plsc.VectorSubcoreMesh(core_axis_name="c", subcore_axis_name="s"),
        out_type=jax.ShapeDtypeStruct((NCORE, NP, wrow), jnp.float32),
        compiler_params=pltpu.CompilerParams(
            needs_layout_passes=False, use_tc_tiling_on_sc=False),
        scratch_types=[
            pltpu.VMEM((CH, d), jnp.float32),
            pltpu.VMEM((CH, d), jnp.float32),
            pltpu.VMEM((CH, wrow), jnp.float32),
            pltpu.VMEM((CH,), jnp.int32),
            pltpu.VMEM((CH,), jnp.int32),
            pltpu.VMEM((CH * 16,), jnp.float32),
            pltpu.VMEM((CH,), jnp.float32),
            pltpu.VMEM((CH,), jnp.float32),
            pltpu.VMEM((d,), jnp.float32),
            pltpu.VMEM_SHARED((NP, wrow), jnp.float32),
            pltpu.SemaphoreType.DMA,
            pltpu.SemaphoreType.DMA,
            pltpu.SemaphoreType.DMA,
        ],
    )
    def sc_pass(xl_hbm, xr_hbm, c_hbm, att_hbm, src_hbm, dst_hbm, out_hbm,
                xlg, xrg, stage, srcv, dstv, pacc, wbuf, cgb, attv, accum,
                sem1, sem2, sem3):
        cid = lax.axis_index("c")
        sid = lax.axis_index("s")
        tid = sid * NCORE + cid

        pltpu.sync_copy(att_hbm, attv)

        def zero_row(i, carry):
            z = jnp.zeros((16,), jnp.float32)
            for s in range(wrow // 16):
                stage[i, pl.ds(s * 16, 16)] = z
            return carry

        lax.fori_loop(0, CH, zero_row, 0)
        for zb in range(ROWS_PER_TILE // CH):
            pltpu.sync_copy(
                stage, accum.at[pl.ds(sid * ROWS_PER_TILE + zb * CH, CH)])
        plsc.subcore_barrier()

        def chunk_body(j, carry):
            base = pl.multiple_of((tid * CPT + j) * CH, CH)
            pltpu.sync_copy(src_hbm.at[pl.ds(base, CH)], srcv)
            pltpu.sync_copy(dst_hbm.at[pl.ds(base, CH)], dstv)
            cp1 = pltpu.async_copy(xl_hbm.at[srcv], xlg, sem1)
            cp2 = pltpu.async_copy(xr_hbm.at[dstv], xrg, sem2)
            cp3 = pltpu.async_copy(c_hbm.at[dstv], cgb, sem3)
            cp1.wait()
            cp2.wait()
            cp3.wait()

            def e_body(i, carry2):
                acc = jnp.zeros((16,), jnp.float32)
                for s in range(nslc):
                    a = xlg[i, pl.ds(s * 16, 16)]
                    bb = xrg[i, pl.ds(s * 16, 16)]
                    z = a + bb
                    lr = jnp.maximum(z, 0.2 * z)
                    acc = acc + lr * attv[pl.ds(s * 16, 16)]
                pacc[pl.ds(pl.multiple_of(i * 16, 16), 16)] = acc
                return carry2

            lax.fori_loop(0, CH, e_body, 0)

            lane = lax.iota(jnp.int32, 16)

            def w_body(gi, carry2):
                off = pl.multiple_of(gi * 16, 16)
                ebase = (lane + off) * 16
                ev = jnp.zeros((16,), jnp.float32)
                for l in range(16):
                    ev = ev + plsc.load_gather(pacc, [ebase + l])
                cg = cgb[pl.ds(off, 16)]
                wbuf[pl.ds(off, 16)] = jnp.exp(ev - cg)
                return carry2

            lax.fori_loop(0, CH // 16, w_body, 0)

            def s_body(i, carry2):
                idx = jnp.full((16,), i, jnp.int32)
                wv = plsc.load_gather(wbuf, [idx])
                for s in range(nslc):
                    stage[i, pl.ds(s * 16, 16)] = (
                        xlg[i, pl.ds(s * 16, 16)] * wv)
                stage[i, pl.ds(d, 16)] = wv
                return carry2

            lax.fori_loop(0, CH, s_body, 0)

            pltpu.sync_copy(stage, accum.at[dstv], add=True)
            return carry

        lax.fori_loop(0, CPT, chunk_body, 0)
        plsc.subcore_barrier()

        rows = pl.ds(sid * ROWS_PER_TILE, ROWS_PER_TILE)
        pltpu.sync_copy(accum.at[rows], out_hbm.at[cid].at[rows])

    return sc_pass


_sc_pass_l1 = _make_sc_edge_pass(128)
_sc_pass_l2 = _make_sc_edge_pass(64)


def kernel(x, edge_index, Wl1, Wr1, att1, b1, Wl2, Wr2, att2, b2, Wlin, blin):
    x = x.astype(jnp.float32)
    xp = jnp.pad(x, ((0, NP - N), (0, 0)))
    src = edge_index[0].astype(jnp.int32)
    dst = edge_index[1].astype(jnp.int32)
    npad = EP - E
    pad_dst = (N + (jnp.arange(npad, dtype=jnp.int32) % (NP - N)))
    srcp = jnp.concatenate([src, jnp.zeros((npad,), jnp.int32)])
    dstp = jnp.concatenate([dst, pad_dst])

    xl1, xr1, c1 = _tc_precompute(xp, Wl1, Wr1, att1)
    acc1 = _sc_pass_l1(xl1, xr1, c1.reshape(NP), att1, srcp, dstp)
    xl2, xr2, c2 = _tc_combine_next(acc1[0], acc1[1], xl1, b1, Wl2, Wr2, att2)
    acc2 = _sc_pass_l2(xl2, xr2, c2.reshape(NP), att2, srcp, dstp)
    y = _tc_combine_head(acc2[0], acc2[1], xl2, b2, Wlin.reshape(-1), blin)
    return y.reshape(NP)[:N]


# speedup vs baseline: 9.3007x; 1.1730x over previous
import functools

import jax
import jax.numpy as jnp
from jax import lax
from jax.experimental import pallas as pl
from jax.experimental.pallas import tpu as pltpu
from jax.experimental.pallas import tpu_sc as plsc

N = 10000
NP = 10240
E = 320000
CH = 64
NSUB = 16
NCORE = 2
NWORK = NSUB * NCORE
CPT = 157
EP = NWORK * CPT * CH
ROWS_PER_TILE = NP // NSUB


def _tc_precompute(x, Wl, Wr, att, blk=256):
    n, d_in = x.shape
    d = Wl.shape[1]
    g = n // blk

    def body(x_ref, wl_ref, wr_ref, att_ref, xl_ref, xr_ref, c_ref):
        xb = x_ref[...]
        xl = jnp.dot(xb, wl_ref[...], preferred_element_type=jnp.float32,
                     precision=lax.Precision.HIGHEST)
        xr = jnp.dot(xb, wr_ref[...], preferred_element_type=jnp.float32,
                     precision=lax.Precision.HIGHEST)
        xl_ref[...] = xl
        xr_ref[...] = xr
        z = xl + xr
        lr = jnp.maximum(z, 0.2 * z)
        c = jnp.sum(lr * att_ref[...], axis=1)
        c_ref[...] = c[None, None, :]

    return pl.pallas_call(
        body,
        grid=(g,),
        in_specs=[
            pl.BlockSpec((blk, d_in), lambda i: (i, 0)),
            pl.BlockSpec((d_in, d), lambda i: (0, 0)),
            pl.BlockSpec((d_in, d), lambda i: (0, 0)),
            pl.BlockSpec((1, d), lambda i: (0, 0)),
        ],
        out_specs=[
            pl.BlockSpec((blk, d), lambda i: (i, 0)),
            pl.BlockSpec((blk, d), lambda i: (i, 0)),
            pl.BlockSpec((1, 1, blk), lambda i: (i, 0, 0)),
        ],
        out_shape=[
            jax.ShapeDtypeStruct((n, d), jnp.float32),
            jax.ShapeDtypeStruct((n, d), jnp.float32),
            jax.ShapeDtypeStruct((g, 1, blk), jnp.float32),
        ],
    )(x, Wl, Wr, att.reshape(1, -1))


def _tc_combine_next(acc0, acc1, xl, b, Wl2, Wr2, att2, blk=256):
    n, d = xl.shape
    d2 = Wl2.shape[1]
    g = n // blk
    w = acc0.shape[1]

    def body(a0_ref, a1_ref, xl_ref, b_ref, wl_ref, wr_ref, att_ref,
             xl2_ref, xr2_ref, c2_ref):
        a0 = a0_ref[...]
        a1 = a1_ref[...]
        numer = a0[:, :d] + a1[:, :d] + xl_ref[...]
        denom = a0[:, d:d + 1] + a1[:, d:d + 1] + 1.0
        h = numer / (denom + 1e-16) + b_ref[...]
        h = jnp.maximum(h, 0.01 * h)
        xl2 = jnp.dot(h, wl_ref[...], preferred_element_type=jnp.float32,
                      precision=lax.Precision.HIGHEST)
        xr2 = jnp.dot(h, wr_ref[...], preferred_element_type=jnp.float32,
                      precision=lax.Precision.HIGHEST)
        xl2_ref[...] = xl2
        xr2_ref[...] = xr2
        z = xl2 + xr2
        lr = jnp.maximum(z, 0.2 * z)
        c2 = jnp.sum(lr * att_ref[...], axis=1)
        c2_ref[...] = c2[None, None, :]

    return pl.pallas_call(
        body,
        grid=(g,),
        in_specs=[
            pl.BlockSpec((blk, w), lambda i: (i, 0)),
            pl.BlockSpec((blk, w), lambda i: (i, 0)),
            pl.BlockSpec((blk, d), lambda i: (i, 0)),
            pl.BlockSpec((1, d), lambda i: (0, 0)),
            pl.BlockSpec((d, d2), lambda i: (0, 0)),
            pl.BlockSpec((d, d2), lambda i: (0, 0)),
            pl.BlockSpec((1, d2), lambda i: (0, 0)),
        ],
        out_specs=[
            pl.BlockSpec((blk, d2), lambda i: (i, 0)),
            pl.BlockSpec((blk, d2), lambda i: (i, 0)),
            pl.BlockSpec((1, 1, blk), lambda i: (i, 0, 0)),
        ],
        out_shape=[
            jax.ShapeDtypeStruct((n, d2), jnp.float32),
            jax.ShapeDtypeStruct((n, d2), jnp.float32),
            jax.ShapeDtypeStruct((g, 1, blk), jnp.float32),
        ],
    )(acc0, acc1, xl, b.reshape(1, -1), Wl2, Wr2, att2.reshape(1, -1))


def _tc_combine_head(acc0, acc1, xl, b, wlin, blin, blk=256):
    n, d = xl.shape
    g = n // blk
    w = acc0.shape[1]

    def body(a0_ref, a1_ref, xl_ref, b_ref, wlin_ref, blin_ref, y_ref):
        a0 = a0_ref[...]
        a1 = a1_ref[...]
        dd = w - 16
        numer = a0[:, :d] + a1[:, :d] + xl_ref[:, :d]
        denom = a0[:, dd:dd + 1] + a1[:, dd:dd + 1] + 1.0
        h = numer / (denom + 1e-16) + b_ref[...]
        y = jnp.sum(h * wlin_ref[...], axis=1) + blin_ref[0, 0]
        y_ref[...] = y[None, None, :]

    return pl.pallas_call(
        body,
        grid=(g,),
        in_specs=[
            pl.BlockSpec((blk, w), lambda i: (i, 0)),
            pl.BlockSpec((blk, w), lambda i: (i, 0)),
            pl.BlockSpec((blk, d), lambda i: (i, 0)),
            pl.BlockSpec((1, d), lambda i: (0, 0)),
            pl.BlockSpec((1, d), lambda i: (0, 0)),
            pl.BlockSpec((1, 1), lambda i: (0, 0)),
        ],
        out_specs=pl.BlockSpec((1, 1, blk), lambda i: (i, 0, 0)),
        out_shape=jax.ShapeDtypeStruct((g, 1, blk), jnp.float32),
    )(acc0, acc1, xl, b.reshape(1, -1), wlin.reshape(1, -1),
      blin.reshape(1, 1))


def _pack_rows(x):
    n, d = x.shape
    a = x.reshape(n, d // 32, 2, 16).swapaxes(2, 3)
    b = a.astype(jnp.bfloat16)
    w = lax.bitcast_convert_type(b, jnp.float32)
    return w.reshape(n, d // 2)


def _make_sc_edge_pass(d):
    wrow = d + 16
    dh = d // 2
    nsl32 = d // 32

    @functools.partial(
        pl.kernel,
        mesh=plsc.VectorSubcoreMesh(core_axis_name="c", subcore_axis_name="s"),
        out_type=jax.ShapeDtypeStruct((NCORE, NP, wrow), jnp.float32),
        compiler_params=pltpu.CompilerParams(
            needs_layout_passes=False, use_tc_tiling_on_sc=False),
        scratch_types=[
            pltpu.VMEM((CH, dh), jnp.float32),
            pltpu.VMEM((CH, dh), jnp.float32),
            pltpu.VMEM((CH, wrow), jnp.float32),
            pltpu.VMEM((CH,), jnp.int32),
            pltpu.VMEM((CH,), jnp.int32),
            pltpu.VMEM((CH * 16,), jnp.float32),
            pltpu.VMEM((CH,), jnp.float32),
            pltpu.VMEM((CH,), jnp.float32),
            pltpu.VMEM((d,), jnp.float32),
            pltpu.VMEM_SHARED((NP, wrow), jnp.float32),
            pltpu.SemaphoreType.DMA,
            pltpu.SemaphoreType.DMA,
            pltpu.SemaphoreType.DMA,
        ],
    )
    def sc_pass(xl_hbm, xr_hbm, c_hbm, att_hbm, src_hbm, dst_hbm, out_hbm,
                xlg, xrg, stage, srcv, dstv, pacc, wbuf, cgb, attv, accum,
                sem1, sem2, sem3):
        cid = lax.axis_index("c")
        sid = lax.axis_index("s")
        tid = sid * NCORE + cid

        pltpu.sync_copy(att_hbm, attv)

        def zero_row(i, carry):
            z = jnp.zeros((16,), jnp.float32)
            for s in range(wrow // 16):
                stage[i, pl.ds(s * 16, 16)] = z
            return carry

        lax.fori_loop(0, CH, zero_row, 0)
        for zb in range(ROWS_PER_TILE // CH):
            pltpu.sync_copy(
                stage, accum.at[pl.ds(sid * ROWS_PER_TILE + zb * CH, CH)])
        plsc.subcore_barrier()

        def chunk_body(j, carry):
            base = pl.multiple_of((tid * CPT + j) * CH, CH)
            pltpu.sync_copy(src_hbm.at[pl.ds(base, CH)], srcv)
            pltpu.sync_copy(dst_hbm.at[pl.ds(base, CH)], dstv)
            cp1 = pltpu.async_copy(xl_hbm.at[srcv], xlg, sem1)
            cp2 = pltpu.async_copy(xr_hbm.at[dstv], xrg, sem2)
            cp3 = pltpu.async_copy(c_hbm.at[dstv], cgb, sem3)
            cp1.wait()
            cp2.wait()
            cp3.wait()

            def e_body(i, carry2):
                acc = jnp.zeros((16,), jnp.float32)
                for s in range(nsl32):
                    aw = plsc.bitcast(xlg[i, pl.ds(s * 16, 16)], jnp.bfloat16)
                    bw = plsc.bitcast(xrg[i, pl.ds(s * 16, 16)], jnp.bfloat16)
                    ae, ao = plsc.unpack(
                        aw, format=plsc.PackFormat.INTERLEAVED,
                        preferred_element_type=jnp.float32)
                    be, bo = plsc.unpack(
                        bw, format=plsc.PackFormat.INTERLEAVED,
                        preferred_element_type=jnp.float32)
                    ze = ae + be
                    zo = ao + bo
                    lre = jnp.maximum(ze, 0.2 * ze)
                    lro = jnp.maximum(zo, 0.2 * zo)
                    acc = (acc + lre * attv[pl.ds(s * 32, 16)]
                           + lro * attv[pl.ds(s * 32 + 16, 16)])
                pacc[pl.ds(pl.multiple_of(i * 16, 16), 16)] = acc
                return carry2

            lax.fori_loop(0, CH, e_body, 0)

            lane = lax.iota(jnp.int32, 16)

            def w_body(gi, carry2):
                off = pl.multiple_of(gi * 16, 16)
                ebase = (lane + off) * 16
                ev = jnp.zeros((16,), jnp.float32)
                for l in range(16):
                    ev = ev + plsc.load_gather(pacc, [ebase + l])
                cg = cgb[pl.ds(off, 16)]
                wbuf[pl.ds(off, 16)] = jnp.exp(ev - cg)
                return carry2

            lax.fori_loop(0, CH // 16, w_body, 0)

            def s_body(i, carry2):
                idx = jnp.full((16,), i, jnp.int32)
                wv = plsc.load_gather(wbuf, [idx])
                for s in range(nsl32):
                    aw = plsc.bitcast(xlg[i, pl.ds(s * 16, 16)], jnp.bfloat16)
                    ae, ao = plsc.unpack(
                        aw, format=plsc.PackFormat.INTERLEAVED,
                        preferred_element_type=jnp.float32)
                    stage[i, pl.ds(s * 32, 16)] = ae * wv
                    stage[i, pl.ds(s * 32 + 16, 16)] = ao * wv
                stage[i, pl.ds(d, 16)] = wv
                return carry2

            lax.fori_loop(0, CH, s_body, 0)

            pltpu.sync_copy(stage, accum.at[dstv], add=True)
            return carry

        lax.fori_loop(0, CPT, chunk_body, 0)
        plsc.subcore_barrier()

        rows = pl.ds(sid * ROWS_PER_TILE, ROWS_PER_TILE)
        pltpu.sync_copy(accum.at[rows], out_hbm.at[cid].at[rows])

    return sc_pass


_sc_pass_l1 = _make_sc_edge_pass(128)
_sc_pass_l2 = _make_sc_edge_pass(64)


def kernel(x, edge_index, Wl1, Wr1, att1, b1, Wl2, Wr2, att2, b2, Wlin, blin):
    x = x.astype(jnp.float32)
    xp = jnp.pad(x, ((0, NP - N), (0, 0)))
    src = edge_index[0].astype(jnp.int32)
    dst = edge_index[1].astype(jnp.int32)
    npad = EP - E
    pad_dst = (N + (jnp.arange(npad, dtype=jnp.int32) % (NP - N)))
    srcp = jnp.concatenate([src, jnp.zeros((npad,), jnp.int32)])
    dstp = jnp.concatenate([dst, pad_dst])

    xl1, xr1, c1 = _tc_precompute(xp, Wl1, Wr1, att1)
    acc1 = _sc_pass_l1(_pack_rows(xl1), _pack_rows(xr1), c1.reshape(NP),
                       att1, srcp, dstp)
    xl2, xr2, c2 = _tc_combine_next(acc1[0], acc1[1], xl1, b1, Wl2, Wr2, att2)
    acc2 = _sc_pass_l2(_pack_rows(xl2), _pack_rows(xr2), c2.reshape(NP),
                       att2, srcp, dstp)
    y = _tc_combine_head(acc2[0], acc2[1], xl2, b2, Wlin.reshape(-1), blin)
    return y.reshape(NP)[:N]


# speedup vs baseline: 10.0734x; 1.0831x over previous
import functools

import jax
import jax.numpy as jnp
from jax import lax
from jax.experimental import pallas as pl
from jax.experimental.pallas import tpu as pltpu
from jax.experimental.pallas import tpu_sc as plsc

N = 10000
NP = 10240
E = 320000
CH = 128
NSUB = 16
NCORE = 2
NWORK = NSUB * NCORE
CPT = 79
EP = NWORK * CPT * CH
ROWS_PER_TILE = NP // NSUB


def _tc_precompute(x, Wl, Wr, att, blk=256):
    n, d_in = x.shape
    d = Wl.shape[1]
    g = n // blk

    def body(x_ref, wl_ref, wr_ref, att_ref, xl_ref, xr_ref, c_ref):
        xb = x_ref[...]
        xl = jnp.dot(xb, wl_ref[...], preferred_element_type=jnp.float32,
                     precision=lax.Precision.HIGHEST)
        xr = jnp.dot(xb, wr_ref[...], preferred_element_type=jnp.float32,
                     precision=lax.Precision.HIGHEST)
        xl_ref[...] = xl
        xr_ref[...] = xr
        z = xl + xr
        lr = jnp.maximum(z, 0.2 * z)
        c = jnp.sum(lr * att_ref[...], axis=1)
        c_ref[...] = c[None, None, :]

    return pl.pallas_call(
        body,
        grid=(g,),
        in_specs=[
            pl.BlockSpec((blk, d_in), lambda i: (i, 0)),
            pl.BlockSpec((d_in, d), lambda i: (0, 0)),
            pl.BlockSpec((d_in, d), lambda i: (0, 0)),
            pl.BlockSpec((1, d), lambda i: (0, 0)),
        ],
        out_specs=[
            pl.BlockSpec((blk, d), lambda i: (i, 0)),
            pl.BlockSpec((blk, d), lambda i: (i, 0)),
            pl.BlockSpec((1, 1, blk), lambda i: (i, 0, 0)),
        ],
        out_shape=[
            jax.ShapeDtypeStruct((n, d), jnp.float32),
            jax.ShapeDtypeStruct((n, d), jnp.float32),
            jax.ShapeDtypeStruct((g, 1, blk), jnp.float32),
        ],
    )(x, Wl, Wr, att.reshape(1, -1))


def _tc_combine_next(acc0, acc1, xl, b, Wl2, Wr2, att2, blk=256):
    n, d = xl.shape
    d2 = Wl2.shape[1]
    g = n // blk
    w = acc0.shape[1]

    def body(a0_ref, a1_ref, xl_ref, b_ref, wl_ref, wr_ref, att_ref,
             xl2_ref, xr2_ref, c2_ref):
        a0 = a0_ref[...]
        a1 = a1_ref[...]
        numer = a0[:, :d] + a1[:, :d] + xl_ref[...]
        denom = a0[:, d:d + 1] + a1[:, d:d + 1] + 1.0
        h = numer / (denom + 1e-16) + b_ref[...]
        h = jnp.maximum(h, 0.01 * h)
        xl2 = jnp.dot(h, wl_ref[...], preferred_element_type=jnp.float32,
                      precision=lax.Precision.HIGHEST)
        xr2 = jnp.dot(h, wr_ref[...], preferred_element_type=jnp.float32,
                      precision=lax.Precision.HIGHEST)
        xl2_ref[...] = xl2
        xr2_ref[...] = xr2
        z = xl2 + xr2
        lr = jnp.maximum(z, 0.2 * z)
        c2 = jnp.sum(lr * att_ref[...], axis=1)
        c2_ref[...] = c2[None, None, :]

    return pl.pallas_call(
        body,
        grid=(g,),
        in_specs=[
            pl.BlockSpec((blk, w), lambda i: (i, 0)),
            pl.BlockSpec((blk, w), lambda i: (i, 0)),
            pl.BlockSpec((blk, d), lambda i: (i, 0)),
            pl.BlockSpec((1, d), lambda i: (0, 0)),
            pl.BlockSpec((d, d2), lambda i: (0, 0)),
            pl.BlockSpec((d, d2), lambda i: (0, 0)),
            pl.BlockSpec((1, d2), lambda i: (0, 0)),
        ],
        out_specs=[
            pl.BlockSpec((blk, d2), lambda i: (i, 0)),
            pl.BlockSpec((blk, d2), lambda i: (i, 0)),
            pl.BlockSpec((1, 1, blk), lambda i: (i, 0, 0)),
        ],
        out_shape=[
            jax.ShapeDtypeStruct((n, d2), jnp.float32),
            jax.ShapeDtypeStruct((n, d2), jnp.float32),
            jax.ShapeDtypeStruct((g, 1, blk), jnp.float32),
        ],
    )(acc0, acc1, xl, b.reshape(1, -1), Wl2, Wr2, att2.reshape(1, -1))


def _tc_combine_head(acc0, acc1, xl, b, wlin, blin, blk=256):
    n, d = xl.shape
    g = n // blk
    w = acc0.shape[1]

    def body(a0_ref, a1_ref, xl_ref, b_ref, wlin_ref, blin_ref, y_ref):
        a0 = a0_ref[...]
        a1 = a1_ref[...]
        dd = w - 16
        numer = a0[:, :d] + a1[:, :d] + xl_ref[:, :d]
        denom = a0[:, dd:dd + 1] + a1[:, dd:dd + 1] + 1.0
        h = numer / (denom + 1e-16) + b_ref[...]
        y = jnp.sum(h * wlin_ref[...], axis=1) + blin_ref[0, 0]
        y_ref[...] = y[None, None, :]

    return pl.pallas_call(
        body,
        grid=(g,),
        in_specs=[
            pl.BlockSpec((blk, w), lambda i: (i, 0)),
            pl.BlockSpec((blk, w), lambda i: (i, 0)),
            pl.BlockSpec((blk, d), lambda i: (i, 0)),
            pl.BlockSpec((1, d), lambda i: (0, 0)),
            pl.BlockSpec((1, d), lambda i: (0, 0)),
            pl.BlockSpec((1, 1), lambda i: (0, 0)),
        ],
        out_specs=pl.BlockSpec((1, 1, blk), lambda i: (i, 0, 0)),
        out_shape=jax.ShapeDtypeStruct((g, 1, blk), jnp.float32),
    )(acc0, acc1, xl, b.reshape(1, -1), wlin.reshape(1, -1),
      blin.reshape(1, 1))


def _pack_rows(x):
    n, d = x.shape
    a = x.reshape(n, d // 32, 2, 16).swapaxes(2, 3)
    b = a.astype(jnp.bfloat16)
    w = lax.bitcast_convert_type(b, jnp.float32)
    return w.reshape(n, d // 2)


def _make_sc_edge_pass(d):
    wrow = d + 16
    dh = d // 2
    nsl32 = d // 32

    @functools.partial(
        pl.kernel,
        mesh=plsc.VectorSubcoreMesh(core_axis_name="c", subcore_axis_name="s"),
        out_type=jax.ShapeDtypeStruct((NCORE, NP, wrow), jnp.float32),
        compiler_params=pltpu.CompilerParams(
            needs_layout_passes=False, use_tc_tiling_on_sc=False),
        scratch_types=[
            pltpu.VMEM((CH, dh), jnp.float32),
            pltpu.VMEM((CH, dh), jnp.float32),
            pltpu.VMEM((CH, wrow), jnp.float32),
            pltpu.VMEM((CH,), jnp.int32),
            pltpu.VMEM((CH,), jnp.int32),
            pltpu.VMEM((CH * 16,), jnp.float32),
            pltpu.VMEM((CH,), jnp.float32),
            pltpu.VMEM((CH,), jnp.float32),
            pltpu.VMEM((d,), jnp.float32),
            pltpu.VMEM_SHARED((NP, wrow), jnp.float32),
            pltpu.SemaphoreType.DMA,
            pltpu.SemaphoreType.DMA,
            pltpu.SemaphoreType.DMA,
        ],
    )
    def sc_pass(xl_hbm, xr_hbm, c_hbm, att_hbm, src_hbm, dst_hbm, out_hbm,
                xlg, xrg, stage, srcv, dstv, pacc, wbuf, cgb, attv, accum,
                sem1, sem2, sem3):
        cid = lax.axis_index("c")
        sid = lax.axis_index("s")
        tid = sid * NCORE + cid

        pltpu.sync_copy(att_hbm, attv)

        def zero_row(i, carry):
            z = jnp.zeros((16,), jnp.float32)
            for s in range(wrow // 16):
                stage[i, pl.ds(s * 16, 16)] = z
            return carry

        lax.fori_loop(0, CH, zero_row, 0)
        for zb in range(ROWS_PER_TILE // CH):
            pltpu.sync_copy(
                stage, accum.at[pl.ds(sid * ROWS_PER_TILE + zb * CH, CH)])
        plsc.subcore_barrier()

        def chunk_body(j, carry):
            base = pl.multiple_of((tid * CPT + j) * CH, CH)
            pltpu.sync_copy(src_hbm.at[pl.ds(base, CH)], srcv)
            pltpu.sync_copy(dst_hbm.at[pl.ds(base, CH)], dstv)
            cp1 = pltpu.async_copy(xl_hbm.at[srcv], xlg, sem1)
            cp2 = pltpu.async_copy(xr_hbm.at[dstv], xrg, sem2)
            cp3 = pltpu.async_copy(c_hbm.at[dstv], cgb, sem3)
            cp1.wait()
            cp2.wait()
            cp3.wait()

            def e_body(i, carry2):
                acc = jnp.zeros((16,), jnp.float32)
                for s in range(nsl32):
                    aw = plsc.bitcast(xlg[i, pl.ds(s * 16, 16)], jnp.bfloat16)
                    bw = plsc.bitcast(xrg[i, pl.ds(s * 16, 16)], jnp.bfloat16)
                    ae, ao = plsc.unpack(
                        aw, format=plsc.PackFormat.INTERLEAVED,
                        preferred_element_type=jnp.float32)
                    be, bo = plsc.unpack(
                        bw, format=plsc.PackFormat.INTERLEAVED,
                        preferred_element_type=jnp.float32)
                    ze = ae + be
                    zo = ao + bo
                    lre = jnp.maximum(ze, 0.2 * ze)
                    lro = jnp.maximum(zo, 0.2 * zo)
                    acc = (acc + lre * attv[pl.ds(s * 32, 16)]
                           + lro * attv[pl.ds(s * 32 + 16, 16)])
                pacc[pl.ds(pl.multiple_of(i * 16, 16), 16)] = acc
                return carry2

            lax.fori_loop(0, CH, e_body, 0)

            lane = lax.iota(jnp.int32, 16)

            def w_body(gi, carry2):
                off = pl.multiple_of(gi * 16, 16)
                ebase = (lane + off) * 16
                ev = jnp.zeros((16,), jnp.float32)
                for l in range(16):
                    ev = ev + plsc.load_gather(pacc, [ebase + l])
                cg = cgb[pl.ds(off, 16)]
                wbuf[pl.ds(off, 16)] = jnp.exp(ev - cg)
                return carry2

            lax.fori_loop(0, CH // 16, w_body, 0)

            def s_body(i, carry2):
                idx = jnp.full((16,), i, jnp.int32)
                wv = plsc.load_gather(wbuf, [idx])
                for s in range(nsl32):
                    aw = plsc.bitcast(xlg[i, pl.ds(s * 16, 16)], jnp.bfloat16)
                    ae, ao = plsc.unpack(
                        aw, format=plsc.PackFormat.INTERLEAVED,
                        preferred_element_type=jnp.float32)
                    stage[i, pl.ds(s * 32, 16)] = ae * wv
                    stage[i, pl.ds(s * 32 + 16, 16)] = ao * wv
                stage[i, pl.ds(d, 16)] = wv
                return carry2

            lax.fori_loop(0, CH, s_body, 0)

            pltpu.sync_copy(stage, accum.at[dstv], add=True)
            return carry

        lax.fori_loop(0, CPT, chunk_body, 0)
        plsc.subcore_barrier()

        rows = pl.ds(sid * ROWS_PER_TILE, ROWS_PER_TILE)
        pltpu.sync_copy(accum.at[rows], out_hbm.at[cid].at[rows])

    return sc_pass


_sc_pass_l1 = _make_sc_edge_pass(128)
_sc_pass_l2 = _make_sc_edge_pass(64)


def kernel(x, edge_index, Wl1, Wr1, att1, b1, Wl2, Wr2, att2, b2, Wlin, blin):
    x = x.astype(jnp.float32)
    xp = jnp.pad(x, ((0, NP - N), (0, 0)))
    src = edge_index[0].astype(jnp.int32)
    dst = edge_index[1].astype(jnp.int32)
    npad = EP - E
    pad_dst = (N + (jnp.arange(npad, dtype=jnp.int32) % (NP - N)))
    srcp = jnp.concatenate([src, jnp.zeros((npad,), jnp.int32)])
    dstp = jnp.concatenate([dst, pad_dst])

    xl1, xr1, c1 = _tc_precompute(xp, Wl1, Wr1, att1)
    acc1 = _sc_pass_l1(_pack_rows(xl1), _pack_rows(xr1), c1.reshape(NP),
                       att1, srcp, dstp)
    xl2, xr2, c2 = _tc_combine_next(acc1[0], acc1[1], xl1, b1, Wl2, Wr2, att2)
    acc2 = _sc_pass_l2(_pack_rows(xl2), _pack_rows(xr2), c2.reshape(NP),
                       att2, srcp, dstp)
    y = _tc_combine_head(acc2[0], acc2[1], xl2, b2, Wlin.reshape(-1), blin)
    return y.reshape(NP)[:N]


# speedup vs baseline: 10.9647x; 1.0885x over previous
import functools

import jax
import jax.numpy as jnp
from jax import lax
from jax.experimental import pallas as pl
from jax.experimental.pallas import tpu as pltpu
from jax.experimental.pallas import tpu_sc as plsc

N = 10000
NP = 10240
E = 320000
CH = 64
NSUB = 16
NCORE = 2
NWORK = NSUB * NCORE
CPT = 158
EP = NWORK * CPT * CH
ROWS_PER_TILE = NP // NSUB


def _tc_precompute(x, Wl, Wr, att, blk=256):
    n, d_in = x.shape
    d = Wl.shape[1]
    g = n // blk

    def body(x_ref, wl_ref, wr_ref, att_ref, xl_ref, xr_ref, c_ref):
        xb = x_ref[...]
        xl = jnp.dot(xb, wl_ref[...], preferred_element_type=jnp.float32,
                     precision=lax.Precision.HIGHEST)
        xr = jnp.dot(xb, wr_ref[...], preferred_element_type=jnp.float32,
                     precision=lax.Precision.HIGHEST)
        xl_ref[...] = xl
        xr_ref[...] = xr
        z = xl + xr
        lr = jnp.maximum(z, 0.2 * z)
        c = jnp.sum(lr * att_ref[...], axis=1)
        c_ref[...] = c[None, None, :]

    return pl.pallas_call(
        body,
        grid=(g,),
        in_specs=[
            pl.BlockSpec((blk, d_in), lambda i: (i, 0)),
            pl.BlockSpec((d_in, d), lambda i: (0, 0)),
            pl.BlockSpec((d_in, d), lambda i: (0, 0)),
            pl.BlockSpec((1, d), lambda i: (0, 0)),
        ],
        out_specs=[
            pl.BlockSpec((blk, d), lambda i: (i, 0)),
            pl.BlockSpec((blk, d), lambda i: (i, 0)),
            pl.BlockSpec((1, 1, blk), lambda i: (i, 0, 0)),
        ],
        out_shape=[
            jax.ShapeDtypeStruct((n, d), jnp.float32),
            jax.ShapeDtypeStruct((n, d), jnp.float32),
            jax.ShapeDtypeStruct((g, 1, blk), jnp.float32),
        ],
    )(x, Wl, Wr, att.reshape(1, -1))


def _tc_combine_next(acc0, acc1, xl, b, Wl2, Wr2, att2, blk=256):
    n, d = xl.shape
    d2 = Wl2.shape[1]
    g = n // blk
    w = acc0.shape[1]

    def body(a0_ref, a1_ref, xl_ref, b_ref, wl_ref, wr_ref, att_ref,
             xl2_ref, xr2_ref, c2_ref):
        a0 = a0_ref[...]
        a1 = a1_ref[...]
        numer = a0[:, :d] + a1[:, :d] + xl_ref[...]
        denom = a0[:, d:d + 1] + a1[:, d:d + 1] + 1.0
        h = numer / (denom + 1e-16) + b_ref[...]
        h = jnp.maximum(h, 0.01 * h)
        xl2 = jnp.dot(h, wl_ref[...], preferred_element_type=jnp.float32,
                      precision=lax.Precision.HIGHEST)
        xr2 = jnp.dot(h, wr_ref[...], preferred_element_type=jnp.float32,
                      precision=lax.Precision.HIGHEST)
        xl2_ref[...] = xl2
        xr2_ref[...] = xr2
        z = xl2 + xr2
        lr = jnp.maximum(z, 0.2 * z)
        c2 = jnp.sum(lr * att_ref[...], axis=1)
        c2_ref[...] = c2[None, None, :]

    return pl.pallas_call(
        body,
        grid=(g,),
        in_specs=[
            pl.BlockSpec((blk, w), lambda i: (i, 0)),
            pl.BlockSpec((blk, w), lambda i: (i, 0)),
            pl.BlockSpec((blk, d), lambda i: (i, 0)),
            pl.BlockSpec((1, d), lambda i: (0, 0)),
            pl.BlockSpec((d, d2), lambda i: (0, 0)),
            pl.BlockSpec((d, d2), lambda i: (0, 0)),
            pl.BlockSpec((1, d2), lambda i: (0, 0)),
        ],
        out_specs=[
            pl.BlockSpec((blk, d2), lambda i: (i, 0)),
            pl.BlockSpec((blk, d2), lambda i: (i, 0)),
            pl.BlockSpec((1, 1, blk), lambda i: (i, 0, 0)),
        ],
        out_shape=[
            jax.ShapeDtypeStruct((n, d2), jnp.float32),
            jax.ShapeDtypeStruct((n, d2), jnp.float32),
            jax.ShapeDtypeStruct((g, 1, blk), jnp.float32),
        ],
    )(acc0, acc1, xl, b.reshape(1, -1), Wl2, Wr2, att2.reshape(1, -1))


def _tc_combine_head(acc0, acc1, xl, b, wlin, blin, blk=256):
    n, d = xl.shape
    g = n // blk
    w = acc0.shape[1]

    def body(a0_ref, a1_ref, xl_ref, b_ref, wlin_ref, blin_ref, y_ref):
        a0 = a0_ref[...]
        a1 = a1_ref[...]
        dd = w - 16
        numer = a0[:, :d] + a1[:, :d] + xl_ref[:, :d]
        denom = a0[:, dd:dd + 1] + a1[:, dd:dd + 1] + 1.0
        h = numer / (denom + 1e-16) + b_ref[...]
        y = jnp.sum(h * wlin_ref[...], axis=1) + blin_ref[0, 0]
        y_ref[...] = y[None, None, :]

    return pl.pallas_call(
        body,
        grid=(g,),
        in_specs=[
            pl.BlockSpec((blk, w), lambda i: (i, 0)),
            pl.BlockSpec((blk, w), lambda i: (i, 0)),
            pl.BlockSpec((blk, d), lambda i: (i, 0)),
            pl.BlockSpec((1, d), lambda i: (0, 0)),
            pl.BlockSpec((1, d), lambda i: (0, 0)),
            pl.BlockSpec((1, 1), lambda i: (0, 0)),
        ],
        out_specs=pl.BlockSpec((1, 1, blk), lambda i: (i, 0, 0)),
        out_shape=jax.ShapeDtypeStruct((g, 1, blk), jnp.float32),
    )(acc0, acc1, xl, b.reshape(1, -1), wlin.reshape(1, -1),
      blin.reshape(1, 1))


def _pack_rows(x):
    n, d = x.shape
    a = x.reshape(n, d // 32, 2, 16).swapaxes(2, 3)
    b = a.astype(jnp.bfloat16)
    w = lax.bitcast_convert_type(b, jnp.float32)
    return w.reshape(n, d // 2)


def _make_sc_edge_pass(d):
    wrow = d + 16
    dh = d // 2
    nsl32 = d // 32
    buf = lambda: [
        pltpu.VMEM((CH, dh), jnp.float32),
        pltpu.VMEM((CH, dh), jnp.float32),
        pltpu.VMEM((CH,), jnp.float32),
        pltpu.VMEM((CH,), jnp.int32),
        pltpu.VMEM((CH,), jnp.int32),
        pltpu.SemaphoreType.DMA,
    ]

    @functools.partial(
        pl.kernel,
        mesh=plsc.VectorSubcoreMesh(core_axis_name="c", subcore_axis_name="s"),
        out_type=jax.ShapeDtypeStruct((NCORE, NP, wrow), jnp.float32),
        compiler_params=pltpu.CompilerParams(
            needs_layout_passes=False, use_tc_tiling_on_sc=False),
        scratch_types=buf() + buf() + [
            pltpu.VMEM((CH, wrow), jnp.float32),
            pltpu.VMEM((CH * 16,), jnp.float32),
            pltpu.VMEM((CH,), jnp.float32),
            pltpu.VMEM((d,), jnp.float32),
            pltpu.VMEM_SHARED((NP, wrow), jnp.float32),
        ],
    )
    def sc_pass(xl_hbm, xr_hbm, c_hbm, att_hbm, src_hbm, dst_hbm, out_hbm,
                xlgA, xrgA, cgbA, srcA, dstA, semA,
                xlgB, xrgB, cgbB, srcB, dstB, semB,
                stage, pacc, wbuf, attv, accum):
        cid = lax.axis_index("c")
        sid = lax.axis_index("s")
        tid = sid * NCORE + cid

        pltpu.sync_copy(att_hbm, attv)

        def zero_row(i, carry):
            z = jnp.zeros((16,), jnp.float32)
            for s in range(wrow // 16):
                stage[i, pl.ds(s * 16, 16)] = z
            return carry

        lax.fori_loop(0, CH, zero_row, 0)
        for zb in range(ROWS_PER_TILE // CH):
            pltpu.sync_copy(
                stage, accum.at[pl.ds(sid * ROWS_PER_TILE + zb * CH, CH)])
        plsc.subcore_barrier()

        def load_idx(j, srcv, dstv):
            base = pl.multiple_of((tid * CPT + j) * CH, CH)
            pltpu.sync_copy(src_hbm.at[pl.ds(base, CH)], srcv)
            pltpu.sync_copy(dst_hbm.at[pl.ds(base, CH)], dstv)

        def fire(xlg, xrg, cgb, srcv, dstv, sem):
            pltpu.async_copy(xl_hbm.at[srcv], xlg, sem)
            pltpu.async_copy(xr_hbm.at[dstv], xrg, sem)
            pltpu.async_copy(c_hbm.at[dstv], cgb, sem)

        def drain(xlg, xrg, cgb, srcv, dstv, sem):
            pltpu.make_async_copy(xl_hbm.at[srcv], xlg, sem).wait()
            pltpu.make_async_copy(xr_hbm.at[dstv], xrg, sem).wait()
            pltpu.make_async_copy(c_hbm.at[dstv], cgb, sem).wait()

        def compute(xlg, xrg, cgb, dstv):
            def e_body(i, carry2):
                acc = jnp.zeros((16,), jnp.float32)
                for s in range(nsl32):
                    aw = plsc.bitcast(xlg[i, pl.ds(s * 16, 16)], jnp.bfloat16)
                    bw = plsc.bitcast(xrg[i, pl.ds(s * 16, 16)], jnp.bfloat16)
                    ae, ao = plsc.unpack(
                        aw, format=plsc.PackFormat.INTERLEAVED,
                        preferred_element_type=jnp.float32)
                    be, bo = plsc.unpack(
                        bw, format=plsc.PackFormat.INTERLEAVED,
                        preferred_element_type=jnp.float32)
                    ze = ae + be
                    zo = ao + bo
                    lre = jnp.maximum(ze, 0.2 * ze)
                    lro = jnp.maximum(zo, 0.2 * zo)
                    acc = (acc + lre * attv[pl.ds(s * 32, 16)]
                           + lro * attv[pl.ds(s * 32 + 16, 16)])
                pacc[pl.ds(pl.multiple_of(i * 16, 16), 16)] = acc
                return carry2

            lax.fori_loop(0, CH, e_body, 0)

            lane = lax.iota(jnp.int32, 16)

            def w_body(gi, carry2):
                off = pl.multiple_of(gi * 16, 16)
                ebase = (lane + off) * 16
                ev = jnp.zeros((16,), jnp.float32)
                for l in range(16):
                    ev = ev + plsc.load_gather(pacc, [ebase + l])
                cg = cgb[pl.ds(off, 16)]
                wbuf[pl.ds(off, 16)] = jnp.exp(ev - cg)
                return carry2

            lax.fori_loop(0, CH // 16, w_body, 0)

            def s_body(i, carry2):
                idx = jnp.full((16,), i, jnp.int32)
                wv = plsc.load_gather(wbuf, [idx])
                for s in range(nsl32):
                    aw = plsc.bitcast(xlg[i, pl.ds(s * 16, 16)], jnp.bfloat16)
                    ae, ao = plsc.unpack(
                        aw, format=plsc.PackFormat.INTERLEAVED,
                        preferred_element_type=jnp.float32)
                    stage[i, pl.ds(s * 32, 16)] = ae * wv
                    stage[i, pl.ds(s * 32 + 16, 16)] = ao * wv
                stage[i, pl.ds(d, 16)] = wv
                return carry2

            lax.fori_loop(0, CH, s_body, 0)

            pltpu.sync_copy(stage, accum.at[dstv], add=True)

        A = (xlgA, xrgA, cgbA, srcA, dstA, semA)
        B = (xlgB, xrgB, cgbB, srcB, dstB, semB)

        load_idx(0, srcA, dstA)
        fire(*A)

        def pair_body(t, carry):
            load_idx(2 * t + 1, srcB, dstB)
            fire(*B)
            drain(*A)
            compute(xlgA, xrgA, cgbA, dstA)
            load_idx(2 * t + 2, srcA, dstA)
            fire(*A)
            drain(*B)
            compute(xlgB, xrgB, cgbB, dstB)
            return carry

        lax.fori_loop(0, CPT // 2 - 1, pair_body, 0)
        load_idx(CPT - 1, srcB, dstB)
        fire(*B)
        drain(*A)
        compute(xlgA, xrgA, cgbA, dstA)
        drain(*B)
        compute(xlgB, xrgB, cgbB, dstB)
        plsc.subcore_barrier()

        rows = pl.ds(sid * ROWS_PER_TILE, ROWS_PER_TILE)
        pltpu.sync_copy(accum.at[rows], out_hbm.at[cid].at[rows])

    return sc_pass


_sc_pass_l1 = _make_sc_edge_pass(128)
_sc_pass_l2 = _make_sc_edge_pass(64)


def kernel(x, edge_index, Wl1, Wr1, att1, b1, Wl2, Wr2, att2, b2, Wlin, blin):
    x = x.astype(jnp.float32)
    xp = jnp.pad(x, ((0, NP - N), (0, 0)))
    src = edge_index[0].astype(jnp.int32)
    dst = edge_index[1].astype(jnp.int32)
    npad = EP - E
    pad_dst = (N + (jnp.arange(npad, dtype=jnp.int32) % (NP - N)))
    srcp = jnp.concatenate([src, jnp.zeros((npad,), jnp.int32)])
    dstp = jnp.concatenate([dst, pad_dst])

    xl1, xr1, c1 = _tc_precompute(xp, Wl1, Wr1, att1)
    acc1 = _sc_pass_l1(_pack_rows(xl1), _pack_rows(xr1), c1.reshape(NP),
                       att1, srcp, dstp)
    xl2, xr2, c2 = _tc_combine_next(acc1[0], acc1[1], xl1, b1, Wl2, Wr2, att2)
    acc2 = _sc_pass_l2(_pack_rows(xl2), _pack_rows(xr2), c2.reshape(NP),
                       att2, srcp, dstp)
    y = _tc_combine_head(acc2[0], acc2[1], xl2, b2, Wlin.reshape(-1), blin)
    return y.reshape(NP)[:N]


# speedup vs baseline: 15.5775x; 1.4207x over previous
import functools

import jax
import jax.numpy as jnp
from jax import lax
from jax.experimental import pallas as pl
from jax.experimental.pallas import tpu as pltpu
from jax.experimental.pallas import tpu_sc as plsc

N = 10000
NP = 10240
E = 320000
CH = 64
NSUB = 16
NCORE = 2
NWORK = NSUB * NCORE
CPT = 158
EP = NWORK * CPT * CH
ROWS_PER_TILE = NP // NSUB


def _tc_precompute(x, Wl, Wr, att, blk=256):
    n, d_in = x.shape
    d = Wl.shape[1]
    g = n // blk

    def body(x_ref, wl_ref, wr_ref, att_ref, xl_ref, xr_ref, c_ref):
        xb = x_ref[...]
        xl = jnp.dot(xb, wl_ref[...], preferred_element_type=jnp.float32,
                     precision=lax.Precision.HIGHEST)
        xr = jnp.dot(xb, wr_ref[...], preferred_element_type=jnp.float32,
                     precision=lax.Precision.HIGHEST)
        xl_ref[...] = xl
        xr_ref[...] = xr
        z = xl + xr
        lr = jnp.maximum(z, 0.2 * z)
        c = jnp.sum(lr * att_ref[...], axis=1)
        c_ref[...] = c[None, None, :]

    return pl.pallas_call(
        body,
        grid=(g,),
        in_specs=[
            pl.BlockSpec((blk, d_in), lambda i: (i, 0)),
            pl.BlockSpec((d_in, d), lambda i: (0, 0)),
            pl.BlockSpec((d_in, d), lambda i: (0, 0)),
            pl.BlockSpec((1, d), lambda i: (0, 0)),
        ],
        out_specs=[
            pl.BlockSpec((blk, d), lambda i: (i, 0)),
            pl.BlockSpec((blk, d), lambda i: (i, 0)),
            pl.BlockSpec((1, 1, blk), lambda i: (i, 0, 0)),
        ],
        out_shape=[
            jax.ShapeDtypeStruct((n, d), jnp.float32),
            jax.ShapeDtypeStruct((n, d), jnp.float32),
            jax.ShapeDtypeStruct((g, 1, blk), jnp.float32),
        ],
    )(x, Wl, Wr, att.reshape(1, -1))


def _tc_combine_next(acc0, acc1, xl, b, Wl2, Wr2, att2, blk=256):
    n, d = xl.shape
    d2 = Wl2.shape[1]
    g = n // blk
    w = acc0.shape[1]

    def body(a0_ref, a1_ref, xl_ref, b_ref, wl_ref, wr_ref, att_ref,
             xl2_ref, xr2_ref, c2_ref):
        a0 = a0_ref[...]
        a1 = a1_ref[...]
        numer = a0[:, :d] + a1[:, :d] + xl_ref[...]
        denom = a0[:, d:d + 1] + a1[:, d:d + 1] + 1.0
        h = numer / (denom + 1e-16) + b_ref[...]
        h = jnp.maximum(h, 0.01 * h)
        xl2 = jnp.dot(h, wl_ref[...], preferred_element_type=jnp.float32,
                      precision=lax.Precision.HIGHEST)
        xr2 = jnp.dot(h, wr_ref[...], preferred_element_type=jnp.float32,
                      precision=lax.Precision.HIGHEST)
        xl2_ref[...] = xl2
        xr2_ref[...] = xr2
        z = xl2 + xr2
        lr = jnp.maximum(z, 0.2 * z)
        c2 = jnp.sum(lr * att_ref[...], axis=1)
        c2_ref[...] = c2[None, None, :]

    return pl.pallas_call(
        body,
        grid=(g,),
        in_specs=[
            pl.BlockSpec((blk, w), lambda i: (i, 0)),
            pl.BlockSpec((blk, w), lambda i: (i, 0)),
            pl.BlockSpec((blk, d), lambda i: (i, 0)),
            pl.BlockSpec((1, d), lambda i: (0, 0)),
            pl.BlockSpec((d, d2), lambda i: (0, 0)),
            pl.BlockSpec((d, d2), lambda i: (0, 0)),
            pl.BlockSpec((1, d2), lambda i: (0, 0)),
        ],
        out_specs=[
            pl.BlockSpec((blk, d2), lambda i: (i, 0)),
            pl.BlockSpec((blk, d2), lambda i: (i, 0)),
            pl.BlockSpec((1, 1, blk), lambda i: (i, 0, 0)),
        ],
        out_shape=[
            jax.ShapeDtypeStruct((n, d2), jnp.float32),
            jax.ShapeDtypeStruct((n, d2), jnp.float32),
            jax.ShapeDtypeStruct((g, 1, blk), jnp.float32),
        ],
    )(acc0, acc1, xl, b.reshape(1, -1), Wl2, Wr2, att2.reshape(1, -1))


def _tc_combine_head(acc0, acc1, xl, b, wlin, blin, blk=256):
    n, d = xl.shape
    g = n // blk
    w = acc0.shape[1]

    def body(a0_ref, a1_ref, xl_ref, b_ref, wlin_ref, blin_ref, y_ref):
        a0 = a0_ref[...]
        a1 = a1_ref[...]
        dd = w - 16
        numer = a0[:, :d] + a1[:, :d] + xl_ref[:, :d]
        denom = a0[:, dd:dd + 1] + a1[:, dd:dd + 1] + 1.0
        h = numer / (denom + 1e-16) + b_ref[...]
        y = jnp.sum(h * wlin_ref[...], axis=1) + blin_ref[0, 0]
        y_ref[...] = y[None, None, :]

    return pl.pallas_call(
        body,
        grid=(g,),
        in_specs=[
            pl.BlockSpec((blk, w), lambda i: (i, 0)),
            pl.BlockSpec((blk, w), lambda i: (i, 0)),
            pl.BlockSpec((blk, d), lambda i: (i, 0)),
            pl.BlockSpec((1, d), lambda i: (0, 0)),
            pl.BlockSpec((1, d), lambda i: (0, 0)),
            pl.BlockSpec((1, 1), lambda i: (0, 0)),
        ],
        out_specs=pl.BlockSpec((1, 1, blk), lambda i: (i, 0, 0)),
        out_shape=jax.ShapeDtypeStruct((g, 1, blk), jnp.float32),
    )(acc0, acc1, xl, b.reshape(1, -1), wlin.reshape(1, -1),
      blin.reshape(1, 1))


def _pack_rows(x):
    n, d = x.shape
    a = x.reshape(n, d // 32, 2, 16).swapaxes(2, 3)
    b = a.astype(jnp.bfloat16)
    w = lax.bitcast_convert_type(b, jnp.float32)
    return w.reshape(n, d // 2)


def _make_sc_edge_pass(d):
    wrow = d + 16
    dh = d // 2
    nsl32 = d // 32
    buf = lambda: [
        pltpu.VMEM((CH, dh), jnp.float32),
        pltpu.VMEM((CH, dh), jnp.float32),
        pltpu.VMEM((CH,), jnp.float32),
        pltpu.VMEM((CH,), jnp.int32),
        pltpu.VMEM((CH,), jnp.int32),
        pltpu.SemaphoreType.DMA,
    ]

    @functools.partial(
        pl.kernel,
        mesh=plsc.VectorSubcoreMesh(core_axis_name="c", subcore_axis_name="s"),
        out_type=jax.ShapeDtypeStruct((NCORE, NP, wrow), jnp.float32),
        compiler_params=pltpu.CompilerParams(
            needs_layout_passes=False, use_tc_tiling_on_sc=False),
        scratch_types=buf() + buf() + [
            pltpu.VMEM((CH, wrow), jnp.float32),
            pltpu.VMEM((CH * 16,), jnp.float32),
            pltpu.VMEM((CH,), jnp.float32),
            pltpu.VMEM((d,), jnp.float32),
            pltpu.VMEM_SHARED((NP, wrow), jnp.float32),
        ],
    )
    def sc_pass(xl_hbm, xr_hbm, c_hbm, att_hbm, src_hbm, dst_hbm, out_hbm,
                xlgA, xrgA, cgbA, srcA, dstA, semA,
                xlgB, xrgB, cgbB, srcB, dstB, semB,
                stage, pacc, wbuf, attv, accum):
        cid = lax.axis_index("c")
        sid = lax.axis_index("s")
        tid = sid * NCORE + cid

        pltpu.sync_copy(att_hbm, attv)

        def zero_row(i, carry):
            z = jnp.zeros((16,), jnp.float32)
            for s in range(wrow // 16):
                stage[i, pl.ds(s * 16, 16)] = z
            return carry

        lax.fori_loop(0, CH, zero_row, 0)
        for zb in range(ROWS_PER_TILE // CH):
            pltpu.sync_copy(
                stage, accum.at[pl.ds(sid * ROWS_PER_TILE + zb * CH, CH)])
        plsc.subcore_barrier()

        def load_idx(j, srcv, dstv):
            base = pl.multiple_of((tid * CPT + j) * CH, CH)
            pltpu.sync_copy(src_hbm.at[pl.ds(base, CH)], srcv)
            pltpu.sync_copy(dst_hbm.at[pl.ds(base, CH)], dstv)

        def fire(xlg, xrg, cgb, srcv, dstv, sem):
            pltpu.async_copy(xl_hbm.at[srcv], xlg, sem)
            pltpu.async_copy(xr_hbm.at[dstv], xrg, sem)
            pltpu.async_copy(c_hbm.at[dstv], cgb, sem)

        def drain(xlg, xrg, cgb, srcv, dstv, sem):
            pltpu.make_async_copy(xl_hbm.at[srcv], xlg, sem).wait()
            pltpu.make_async_copy(xr_hbm.at[dstv], xrg, sem).wait()
            pltpu.make_async_copy(c_hbm.at[dstv], cgb, sem).wait()

        def compute(xlg, xrg, cgb, dstv):
            @plsc.parallel_loop(0, CH, unroll=4)
            def e_body(i):
                acc = jnp.zeros((16,), jnp.float32)
                for s in range(nsl32):
                    aw = plsc.bitcast(xlg[i, pl.ds(s * 16, 16)], jnp.bfloat16)
                    bw = plsc.bitcast(xrg[i, pl.ds(s * 16, 16)], jnp.bfloat16)
                    ae, ao = plsc.unpack(
                        aw, format=plsc.PackFormat.INTERLEAVED,
                        preferred_element_type=jnp.float32)
                    be, bo = plsc.unpack(
                        bw, format=plsc.PackFormat.INTERLEAVED,
                        preferred_element_type=jnp.float32)
                    ze = ae + be
                    zo = ao + bo
                    lre = jnp.maximum(ze, 0.2 * ze)
                    lro = jnp.maximum(zo, 0.2 * zo)
                    acc = (acc + lre * attv[pl.ds(s * 32, 16)]
                           + lro * attv[pl.ds(s * 32 + 16, 16)])
                pacc[pl.ds(pl.multiple_of(i * 16, 16), 16)] = acc

            lane = lax.iota(jnp.int32, 16)

            @plsc.parallel_loop(0, CH // 16, unroll=2)
            def w_body(gi):
                off = pl.multiple_of(gi * 16, 16)
                ebase = (lane + off) * 16
                ev = jnp.zeros((16,), jnp.float32)
                for l in range(16):
                    ev = ev + plsc.load_gather(pacc, [ebase + l])
                cg = cgb[pl.ds(off, 16)]
                wbuf[pl.ds(off, 16)] = jnp.exp(ev - cg)

            @plsc.parallel_loop(0, CH, unroll=4)
            def s_body(i):
                idx = jnp.full((16,), i, jnp.int32)
                wv = plsc.load_gather(wbuf, [idx])
                for s in range(nsl32):
                    aw = plsc.bitcast(xlg[i, pl.ds(s * 16, 16)], jnp.bfloat16)
                    ae, ao = plsc.unpack(
                        aw, format=plsc.PackFormat.INTERLEAVED,
                        preferred_element_type=jnp.float32)
                    stage[i, pl.ds(s * 32, 16)] = ae * wv
                    stage[i, pl.ds(s * 32 + 16, 16)] = ao * wv
                stage[i, pl.ds(d, 16)] = wv

            pltpu.sync_copy(stage, accum.at[dstv], add=True)

        A = (xlgA, xrgA, cgbA, srcA, dstA, semA)
        B = (xlgB, xrgB, cgbB, srcB, dstB, semB)

        load_idx(0, srcA, dstA)
        fire(*A)

        def pair_body(t, carry):
            load_idx(2 * t + 1, srcB, dstB)
            fire(*B)
            drain(*A)
            compute(xlgA, xrgA, cgbA, dstA)
            load_idx(2 * t + 2, srcA, dstA)
            fire(*A)
            drain(*B)
            compute(xlgB, xrgB, cgbB, dstB)
            return carry

        lax.fori_loop(0, CPT // 2 - 1, pair_body, 0)
        load_idx(CPT - 1, srcB, dstB)
        fire(*B)
        drain(*A)
        compute(xlgA, xrgA, cgbA, dstA)
        drain(*B)
        compute(xlgB, xrgB, cgbB, dstB)
        plsc.subcore_barrier()

        rows = pl.ds(sid * ROWS_PER_TILE, ROWS_PER_TILE)
        pltpu.sync_copy(accum.at[rows], out_hbm.at[cid].at[rows])

    return sc_pass


_sc_pass_l1 = _make_sc_edge_pass(128)
_sc_pass_l2 = _make_sc_edge_pass(64)


def kernel(x, edge_index, Wl1, Wr1, att1, b1, Wl2, Wr2, att2, b2, Wlin, blin):
    x = x.astype(jnp.float32)
    xp = jnp.pad(x, ((0, NP - N), (0, 0)))
    src = edge_index[0].astype(jnp.int32)
    dst = edge_index[1].astype(jnp.int32)
    npad = EP - E
    pad_dst = (N + (jnp.arange(npad, dtype=jnp.int32) % (NP - N)))
    srcp = jnp.concatenate([src, jnp.zeros((npad,), jnp.int32)])
    dstp = jnp.concatenate([dst, pad_dst])

    xl1, xr1, c1 = _tc_precompute(xp, Wl1, Wr1, att1)
    acc1 = _sc_pass_l1(_pack_rows(xl1), _pack_rows(xr1), c1.reshape(NP),
                       att1, srcp, dstp)
    xl2, xr2, c2 = _tc_combine_next(acc1[0], acc1[1], xl1, b1, Wl2, Wr2, att2)
    acc2 = _sc_pass_l2(_pack_rows(xl2), _pack_rows(xr2), c2.reshape(NP),
                       att2, srcp, dstp)
    y = _tc_combine_head(acc2[0], acc2[1], xl2, b2, Wlin.reshape(-1), blin)
    return y.reshape(NP)[:N]


# speedup vs baseline: 15.7869x; 1.0134x over previous
import functools

import jax
import jax.numpy as jnp
from jax import lax
from jax.experimental import pallas as pl
from jax.experimental.pallas import tpu as pltpu
from jax.experimental.pallas import tpu_sc as plsc

N = 10000
NP = 10240
E = 320000
CH = 64
NSUB = 16
NCORE = 2
NWORK = NSUB * NCORE
CPT = 158
EP = NWORK * CPT * CH
ROWS_PER_TILE = NP // NSUB


def _tc_precompute(x, Wl, Wr, att, blk=256):
    n, d_in = x.shape
    d = Wl.shape[1]
    g = n // blk

    def body(x_ref, wl_ref, wr_ref, att_ref, xl_ref, xr_ref, c_ref):
        xb = x_ref[...]
        xl = jnp.dot(xb, wl_ref[...], preferred_element_type=jnp.float32,
                     precision=lax.Precision.HIGHEST)
        xr = jnp.dot(xb, wr_ref[...], preferred_element_type=jnp.float32,
                     precision=lax.Precision.HIGHEST)
        xl_ref[...] = xl
        xr_ref[...] = xr
        z = xl + xr
        lr = jnp.maximum(z, 0.2 * z)
        c = jnp.sum(lr * att_ref[...], axis=1)
        c_ref[...] = c[None, None, :]

    return pl.pallas_call(
        body,
        grid=(g,),
        in_specs=[
            pl.BlockSpec((blk, d_in), lambda i: (i, 0)),
            pl.BlockSpec((d_in, d), lambda i: (0, 0)),
            pl.BlockSpec((d_in, d), lambda i: (0, 0)),
            pl.BlockSpec((1, d), lambda i: (0, 0)),
        ],
        out_specs=[
            pl.BlockSpec((blk, d), lambda i: (i, 0)),
            pl.BlockSpec((blk, d), lambda i: (i, 0)),
            pl.BlockSpec((1, 1, blk), lambda i: (i, 0, 0)),
        ],
        out_shape=[
            jax.ShapeDtypeStruct((n, d), jnp.float32),
            jax.ShapeDtypeStruct((n, d), jnp.float32),
            jax.ShapeDtypeStruct((g, 1, blk), jnp.float32),
        ],
    )(x, Wl, Wr, att.reshape(1, -1))


def _tc_combine_next(acc0, acc1, xl, b, Wl2, Wr2, att2, blk=256):
    n, d = xl.shape
    d2 = Wl2.shape[1]
    g = n // blk
    w = acc0.shape[1]

    def body(a0_ref, a1_ref, xl_ref, b_ref, wl_ref, wr_ref, att_ref,
             xl2_ref, xr2_ref, c2_ref):
        a0 = a0_ref[...]
        a1 = a1_ref[...]
        numer = a0[:, :d] + a1[:, :d] + xl_ref[...]
        denom = a0[:, d:d + 1] + a1[:, d:d + 1] + 1.0
        h = numer / (denom + 1e-16) + b_ref[...]
        h = jnp.maximum(h, 0.01 * h)
        xl2 = jnp.dot(h, wl_ref[...], preferred_element_type=jnp.float32,
                      precision=lax.Precision.HIGHEST)
        xr2 = jnp.dot(h, wr_ref[...], preferred_element_type=jnp.float32,
                      precision=lax.Precision.HIGHEST)
        xl2_ref[...] = xl2
        xr2_ref[...] = xr2
        z = xl2 + xr2
        lr = jnp.maximum(z, 0.2 * z)
        c2 = jnp.sum(lr * att_ref[...], axis=1)
        c2_ref[...] = c2[None, None, :]

    return pl.pallas_call(
        body,
        grid=(g,),
        in_specs=[
            pl.BlockSpec((blk, w), lambda i: (i, 0)),
            pl.BlockSpec((blk, w), lambda i: (i, 0)),
            pl.BlockSpec((blk, d), lambda i: (i, 0)),
            pl.BlockSpec((1, d), lambda i: (0, 0)),
            pl.BlockSpec((d, d2), lambda i: (0, 0)),
            pl.BlockSpec((d, d2), lambda i: (0, 0)),
            pl.BlockSpec((1, d2), lambda i: (0, 0)),
        ],
        out_specs=[
            pl.BlockSpec((blk, d2), lambda i: (i, 0)),
            pl.BlockSpec((blk, d2), lambda i: (i, 0)),
            pl.BlockSpec((1, 1, blk), lambda i: (i, 0, 0)),
        ],
        out_shape=[
            jax.ShapeDtypeStruct((n, d2), jnp.float32),
            jax.ShapeDtypeStruct((n, d2), jnp.float32),
            jax.ShapeDtypeStruct((g, 1, blk), jnp.float32),
        ],
    )(acc0, acc1, xl, b.reshape(1, -1), Wl2, Wr2, att2.reshape(1, -1))


def _tc_combine_head(acc0, acc1, xl, b, wlin, blin, blk=256):
    n, d = xl.shape
    g = n // blk
    w = acc0.shape[1]

    def body(a0_ref, a1_ref, xl_ref, b_ref, wlin_ref, blin_ref, y_ref):
        a0 = a0_ref[...]
        a1 = a1_ref[...]
        dd = w - 16
        numer = a0[:, :d] + a1[:, :d] + xl_ref[:, :d]
        denom = a0[:, dd:dd + 1] + a1[:, dd:dd + 1] + 1.0
        h = numer / (denom + 1e-16) + b_ref[...]
        y = jnp.sum(h * wlin_ref[...], axis=1) + blin_ref[0, 0]
        y_ref[...] = y[None, None, :]

    return pl.pallas_call(
        body,
        grid=(g,),
        in_specs=[
            pl.BlockSpec((blk, w), lambda i: (i, 0)),
            pl.BlockSpec((blk, w), lambda i: (i, 0)),
            pl.BlockSpec((blk, d), lambda i: (i, 0)),
            pl.BlockSpec((1, d), lambda i: (0, 0)),
            pl.BlockSpec((1, d), lambda i: (0, 0)),
            pl.BlockSpec((1, 1), lambda i: (0, 0)),
        ],
        out_specs=pl.BlockSpec((1, 1, blk), lambda i: (i, 0, 0)),
        out_shape=jax.ShapeDtypeStruct((g, 1, blk), jnp.float32),
    )(acc0, acc1, xl, b.reshape(1, -1), wlin.reshape(1, -1),
      blin.reshape(1, 1))


def _pack_rows(x):
    n, d = x.shape
    a = x.reshape(n, d // 32, 2, 16).swapaxes(2, 3)
    b = a.astype(jnp.bfloat16)
    w = lax.bitcast_convert_type(b, jnp.float32)
    return w.reshape(n, d // 2)


def _make_sc_edge_pass(d):
    wrow = d + 16
    dh = d // 2
    nsl32 = d // 32
    buf = lambda: [
        pltpu.VMEM((CH, dh), jnp.float32),
        pltpu.VMEM((CH, dh), jnp.float32),
        pltpu.VMEM((CH,), jnp.float32),
        pltpu.VMEM((CH,), jnp.int32),
        pltpu.VMEM((CH,), jnp.int32),
        pltpu.SemaphoreType.DMA,
    ]

    @functools.partial(
        pl.kernel,
        mesh=plsc.VectorSubcoreMesh(core_axis_name="c", subcore_axis_name="s"),
        out_type=jax.ShapeDtypeStruct((NCORE, NP, wrow), jnp.float32),
        compiler_params=pltpu.CompilerParams(
            needs_layout_passes=False, use_tc_tiling_on_sc=False),
        scratch_types=buf() + buf() + [
            pltpu.VMEM((CH, wrow), jnp.float32),
            pltpu.VMEM((CH * 16,), jnp.float32),
            pltpu.VMEM((CH,), jnp.float32),
            pltpu.VMEM((d,), jnp.float32),
            pltpu.VMEM_SHARED((NP, wrow), jnp.float32),
        ],
    )
    def sc_pass(xl_hbm, xr_hbm, c_hbm, att_hbm, src_hbm, dst_hbm, out_hbm,
                xlgA, xrgA, cgbA, srcA, dstA, semA,
                xlgB, xrgB, cgbB, srcB, dstB, semB,
                stage, pacc, wbuf, attv, accum):
        cid = lax.axis_index("c")
        sid = lax.axis_index("s")
        tid = sid * NCORE + cid

        pltpu.sync_copy(att_hbm, attv)

        def zero_row(i, carry):
            z = jnp.zeros((16,), jnp.float32)
            for s in range(wrow // 16):
                stage[i, pl.ds(s * 16, 16)] = z
            return carry

        lax.fori_loop(0, CH, zero_row, 0)
        for zb in range(ROWS_PER_TILE // CH):
            pltpu.sync_copy(
                stage, accum.at[pl.ds(sid * ROWS_PER_TILE + zb * CH, CH)])
        plsc.subcore_barrier()

        def load_idx(j, srcv, dstv):
            base = pl.multiple_of((tid * CPT + j) * CH, CH)
            pltpu.sync_copy(src_hbm.at[pl.ds(base, CH)], srcv)
            pltpu.sync_copy(dst_hbm.at[pl.ds(base, CH)], dstv)

        def fire(xlg, xrg, cgb, srcv, dstv, sem):
            pltpu.async_copy(xl_hbm.at[srcv], xlg, sem)
            pltpu.async_copy(xr_hbm.at[dstv], xrg, sem)
            pltpu.async_copy(c_hbm.at[dstv], cgb, sem)

        def drain(xlg, xrg, cgb, srcv, dstv, sem):
            pltpu.make_async_copy(xl_hbm.at[srcv], xlg, sem).wait()
            pltpu.make_async_copy(xr_hbm.at[dstv], xrg, sem).wait()
            pltpu.make_async_copy(c_hbm.at[dstv], cgb, sem).wait()

        def compute(xlg, xrg, cgb, dstv):
            @plsc.parallel_loop(0, CH, unroll=8)
            def e_body(i):
                acc = jnp.zeros((16,), jnp.float32)
                for s in range(nsl32):
                    aw = plsc.bitcast(xlg[i, pl.ds(s * 16, 16)], jnp.bfloat16)
                    bw = plsc.bitcast(xrg[i, pl.ds(s * 16, 16)], jnp.bfloat16)
                    ae, ao = plsc.unpack(
                        aw, format=plsc.PackFormat.INTERLEAVED,
                        preferred_element_type=jnp.float32)
                    be, bo = plsc.unpack(
                        bw, format=plsc.PackFormat.INTERLEAVED,
                        preferred_element_type=jnp.float32)
                    ze = ae + be
                    zo = ao + bo
                    lre = jnp.maximum(ze, 0.2 * ze)
                    lro = jnp.maximum(zo, 0.2 * zo)
                    acc = (acc + lre * attv[pl.ds(s * 32, 16)]
                           + lro * attv[pl.ds(s * 32 + 16, 16)])
                pacc[pl.ds(pl.multiple_of(i * 16, 16), 16)] = acc

            lane = lax.iota(jnp.int32, 16)

            @plsc.parallel_loop(0, CH // 16, unroll=4)
            def w_body(gi):
                off = pl.multiple_of(gi * 16, 16)
                ebase = (lane + off) * 16
                ev = jnp.zeros((16,), jnp.float32)
                for l in range(16):
                    ev = ev + plsc.load_gather(pacc, [ebase + l])
                cg = cgb[pl.ds(off, 16)]
                wbuf[pl.ds(off, 16)] = jnp.exp(ev - cg)

            @plsc.parallel_loop(0, CH, unroll=8)
            def s_body(i):
                idx = jnp.full((16,), i, jnp.int32)
                wv = plsc.load_gather(wbuf, [idx])
                for s in range(nsl32):
                    aw = plsc.bitcast(xlg[i, pl.ds(s * 16, 16)], jnp.bfloat16)
                    ae, ao = plsc.unpack(
                        aw, format=plsc.PackFormat.INTERLEAVED,
                        preferred_element_type=jnp.float32)
                    stage[i, pl.ds(s * 32, 16)] = ae * wv
                    stage[i, pl.ds(s * 32 + 16, 16)] = ao * wv
                stage[i, pl.ds(d, 16)] = wv

            pltpu.sync_copy(stage, accum.at[dstv], add=True)

        A = (xlgA, xrgA, cgbA, srcA, dstA, semA)
        B = (xlgB, xrgB, cgbB, srcB, dstB, semB)

        load_idx(0, srcA, dstA)
        fire(*A)

        def pair_body(t, carry):
            load_idx(2 * t + 1, srcB, dstB)
            fire(*B)
            drain(*A)
            compute(xlgA, xrgA, cgbA, dstA)
            load_idx(2 * t + 2, srcA, dstA)
            fire(*A)
            drain(*B)
            compute(xlgB, xrgB, cgbB, dstB)
            return carry

        lax.fori_loop(0, CPT // 2 - 1, pair_body, 0)
        load_idx(CPT - 1, srcB, dstB)
        fire(*B)
        drain(*A)
        compute(xlgA, xrgA, cgbA, dstA)
        drain(*B)
        compute(xlgB, xrgB, cgbB, dstB)
        plsc.subcore_barrier()

        rows = pl.ds(sid * ROWS_PER_TILE, ROWS_PER_TILE)
        pltpu.sync_copy(accum.at[rows], out_hbm.at[cid].at[rows])

    return sc_pass


_sc_pass_l1 = _make_sc_edge_pass(128)
_sc_pass_l2 = _make_sc_edge_pass(64)


def kernel(x, edge_index, Wl1, Wr1, att1, b1, Wl2, Wr2, att2, b2, Wlin, blin):
    x = x.astype(jnp.float32)
    xp = jnp.pad(x, ((0, NP - N), (0, 0)))
    src = edge_index[0].astype(jnp.int32)
    dst = edge_index[1].astype(jnp.int32)
    npad = EP - E
    pad_dst = (N + (jnp.arange(npad, dtype=jnp.int32) % (NP - N)))
    srcp = jnp.concatenate([src, jnp.zeros((npad,), jnp.int32)])
    dstp = jnp.concatenate([dst, pad_dst])

    xl1, xr1, c1 = _tc_precompute(xp, Wl1, Wr1, att1)
    acc1 = _sc_pass_l1(_pack_rows(xl1), _pack_rows(xr1), c1.reshape(NP),
                       att1, srcp, dstp)
    xl2, xr2, c2 = _tc_combine_next(acc1[0], acc1[1], xl1, b1, Wl2, Wr2, att2)
    acc2 = _sc_pass_l2(_pack_rows(xl2), _pack_rows(xr2), c2.reshape(NP),
                       att2, srcp, dstp)
    y = _tc_combine_head(acc2[0], acc2[1], xl2, b2, Wlin.reshape(-1), blin)
    return y.reshape(NP)[:N]


# speedup vs baseline: 16.3857x; 1.0379x over previous
import functools

import jax
import jax.numpy as jnp
from jax import lax
from jax.experimental import pallas as pl
from jax.experimental.pallas import tpu as pltpu
from jax.experimental.pallas import tpu_sc as plsc

N = 10000
NP = 10240
E = 320000
CH = 64
NSUB = 16
NCORE = 2
NWORK = NSUB * NCORE
CPT = 158
EP = NWORK * CPT * CH
ROWS_PER_TILE = NP // NSUB


def _group_split(d):
    import numpy as np
    cols = np.arange(d).reshape(d // 32, 2, 16)
    return cols[:, 0, :].reshape(-1), cols[:, 1, :].reshape(-1)


def _bf16_round(x):
    return x.astype(jnp.bfloat16).astype(jnp.float32)


def _tc_precompute(x, Wl, Wr, att, blk=256):
    n, d_in = x.shape
    d = Wl.shape[1]
    dh = d // 2
    g = n // blk
    ia, ib = _group_split(d)
    Wla, Wlb = Wl[:, ia], Wl[:, ib]
    Wra, Wrb = Wr[:, ia], Wr[:, ib]
    atta, attb = att[ia].reshape(1, -1), att[ib].reshape(1, -1)

    def body(x_ref, wl_ref, wla_ref, wlb_ref, wra_ref, wrb_ref,
             atta_ref, attb_ref, xl_ref, xlp_ref, xrp_ref, c_ref):
        xb = x_ref[...]
        mm = lambda w: jnp.dot(xb, w, preferred_element_type=jnp.float32,
                               precision=lax.Precision.HIGHEST)
        xl_ref[...] = mm(wl_ref[...])
        xla = _bf16_round(mm(wla_ref[...]))
        xlb = _bf16_round(mm(wlb_ref[...]))
        xra = _bf16_round(mm(wra_ref[...]))
        xrb = _bf16_round(mm(wrb_ref[...]))
        za = xla + xra
        zb = xlb + xrb
        lra = jnp.maximum(za, 0.2 * za)
        lrb = jnp.maximum(zb, 0.2 * zb)
        c = (jnp.sum(lra * atta_ref[...], axis=1)
             + jnp.sum(lrb * attb_ref[...], axis=1))
        c_ref[...] = c[None, None, :]
        xlp_ref[...] = pltpu.pack_elementwise(
            [xla, xlb], packed_dtype=jnp.bfloat16)
        xrp_ref[...] = pltpu.pack_elementwise(
            [xra, xrb], packed_dtype=jnp.bfloat16)

    return pl.pallas_call(
        body,
        grid=(g,),
        in_specs=[
            pl.BlockSpec((blk, d_in), lambda i: (i, 0)),
            pl.BlockSpec((d_in, d), lambda i: (0, 0)),
            pl.BlockSpec((d_in, dh), lambda i: (0, 0)),
            pl.BlockSpec((d_in, dh), lambda i: (0, 0)),
            pl.BlockSpec((d_in, dh), lambda i: (0, 0)),
            pl.BlockSpec((d_in, dh), lambda i: (0, 0)),
            pl.BlockSpec((1, dh), lambda i: (0, 0)),
            pl.BlockSpec((1, dh), lambda i: (0, 0)),
        ],
        out_specs=[
            pl.BlockSpec((blk, d), lambda i: (i, 0)),
            pl.BlockSpec((blk, dh), lambda i: (i, 0)),
            pl.BlockSpec((blk, dh), lambda i: (i, 0)),
            pl.BlockSpec((1, 1, blk), lambda i: (i, 0, 0)),
        ],
        out_shape=[
            jax.ShapeDtypeStruct((n, d), jnp.float32),
            jax.ShapeDtypeStruct((n, dh), jnp.int32),
            jax.ShapeDtypeStruct((n, dh), jnp.int32),
            jax.ShapeDtypeStruct((g, 1, blk), jnp.float32),
        ],
    )(x, Wl, Wla, Wlb, Wra, Wrb, atta, attb)


def _tc_combine_next(acc0, acc1, xl, b, Wl2, Wr2, att2, blk=256):
    n, d = xl.shape
    d2 = Wl2.shape[1]
    d2h = d2 // 2
    g = n // blk
    w = acc0.shape[1]
    ia, ib = _group_split(d2)
    Wla, Wlb = Wl2[:, ia], Wl2[:, ib]
    Wra, Wrb = Wr2[:, ia], Wr2[:, ib]
    atta, attb = att2[ia].reshape(1, -1), att2[ib].reshape(1, -1)

    def body(a0_ref, a1_ref, xl_ref, b_ref, wl_ref, wla_ref, wlb_ref,
             wra_ref, wrb_ref, atta_ref, attb_ref,
             xl2_ref, xlp_ref, xrp_ref, c2_ref):
        a0 = a0_ref[...]
        a1 = a1_ref[...]
        numer = a0[:, :d] + a1[:, :d] + xl_ref[...]
        denom = a0[:, d:d + 1] + a1[:, d:d + 1] + 1.0
        h = numer / (denom + 1e-16) + b_ref[...]
        h = jnp.maximum(h, 0.01 * h)
        mm = lambda w_: jnp.dot(h, w_, preferred_element_type=jnp.float32,
                                precision=lax.Precision.HIGHEST)
        xl2_ref[...] = mm(wl_ref[...])
        xla = _bf16_round(mm(wla_ref[...]))
        xlb = _bf16_round(mm(wlb_ref[...]))
        xra = _bf16_round(mm(wra_ref[...]))
        xrb = _bf16_round(mm(wrb_ref[...]))
        za = xla + xra
        zb = xlb + xrb
        lra = jnp.maximum(za, 0.2 * za)
        lrb = jnp.maximum(zb, 0.2 * zb)
        c2 = (jnp.sum(lra * atta_ref[...], axis=1)
              + jnp.sum(lrb * attb_ref[...], axis=1))
        c2_ref[...] = c2[None, None, :]
        xlp_ref[...] = pltpu.pack_elementwise(
            [xla, xlb], packed_dtype=jnp.bfloat16)
        xrp_ref[...] = pltpu.pack_elementwise(
            [xra, xrb], packed_dtype=jnp.bfloat16)

    return pl.pallas_call(
        body,
        grid=(g,),
        in_specs=[
            pl.BlockSpec((blk, w), lambda i: (i, 0)),
            pl.BlockSpec((blk, w), lambda i: (i, 0)),
            pl.BlockSpec((blk, d), lambda i: (i, 0)),
            pl.BlockSpec((1, d), lambda i: (0, 0)),
            pl.BlockSpec((d, d2), lambda i: (0, 0)),
            pl.BlockSpec((d, d2h), lambda i: (0, 0)),
            pl.BlockSpec((d, d2h), lambda i: (0, 0)),
            pl.BlockSpec((d, d2h), lambda i: (0, 0)),
            pl.BlockSpec((d, d2h), lambda i: (0, 0)),
            pl.BlockSpec((1, d2h), lambda i: (0, 0)),
            pl.BlockSpec((1, d2h), lambda i: (0, 0)),
        ],
        out_specs=[
            pl.BlockSpec((blk, d2), lambda i: (i, 0)),
            pl.BlockSpec((blk, d2h), lambda i: (i, 0)),
            pl.BlockSpec((blk, d2h), lambda i: (i, 0)),
            pl.BlockSpec((1, 1, blk), lambda i: (i, 0, 0)),
        ],
        out_shape=[
            jax.ShapeDtypeStruct((n, d2), jnp.float32),
            jax.ShapeDtypeStruct((n, d2h), jnp.int32),
            jax.ShapeDtypeStruct((n, d2h), jnp.int32),
            jax.ShapeDtypeStruct((g, 1, blk), jnp.float32),
        ],
    )(acc0, acc1, xl, b.reshape(1, -1), Wl2, Wla, Wlb, Wra, Wrb, atta, attb)


def _tc_combine_head(acc0, acc1, xl, b, wlin, blin, blk=256):
    n, d = xl.shape
    g = n // blk
    w = acc0.shape[1]

    def body(a0_ref, a1_ref, xl_ref, b_ref, wlin_ref, blin_ref, y_ref):
        a0 = a0_ref[...]
        a1 = a1_ref[...]
        dd = w - 16
        numer = a0[:, :d] + a1[:, :d] + xl_ref[:, :d]
        denom = a0[:, dd:dd + 1] + a1[:, dd:dd + 1] + 1.0
        h = numer / (denom + 1e-16) + b_ref[...]
        y = jnp.sum(h * wlin_ref[...], axis=1) + blin_ref[0, 0]
        y_ref[...] = y[None, None, :]

    return pl.pallas_call(
        body,
        grid=(g,),
        in_specs=[
            pl.BlockSpec((blk, w), lambda i: (i, 0)),
            pl.BlockSpec((blk, w), lambda i: (i, 0)),
            pl.BlockSpec((blk, d), lambda i: (i, 0)),
            pl.BlockSpec((1, d), lambda i: (0, 0)),
            pl.BlockSpec((1, d), lambda i: (0, 0)),
            pl.BlockSpec((1, 1), lambda i: (0, 0)),
        ],
        out_specs=pl.BlockSpec((1, 1, blk), lambda i: (i, 0, 0)),
        out_shape=jax.ShapeDtypeStruct((g, 1, blk), jnp.float32),
    )(acc0, acc1, xl, b.reshape(1, -1), wlin.reshape(1, -1),
      blin.reshape(1, 1))


def _make_sc_edge_pass(d):
    wrow = d + 16
    dh = d // 2
    nsl32 = d // 32
    buf = lambda: [
        pltpu.VMEM((CH, dh), jnp.int32),
        pltpu.VMEM((CH, dh), jnp.int32),
        pltpu.VMEM((CH,), jnp.float32),
        pltpu.VMEM((CH,), jnp.int32),
        pltpu.VMEM((CH,), jnp.int32),
        pltpu.SemaphoreType.DMA,
    ]

    @functools.partial(
        pl.kernel,
        mesh=plsc.VectorSubcoreMesh(core_axis_name="c", subcore_axis_name="s"),
        out_type=jax.ShapeDtypeStruct((NCORE, NP, wrow), jnp.float32),
        compiler_params=pltpu.CompilerParams(
            needs_layout_passes=False, use_tc_tiling_on_sc=False),
        scratch_types=buf() + buf() + [
            pltpu.VMEM((CH, wrow), jnp.float32),
            pltpu.VMEM((CH * 16,), jnp.float32),
            pltpu.VMEM((CH,), jnp.float32),
            pltpu.VMEM((d,), jnp.float32),
            pltpu.VMEM_SHARED((NP, wrow), jnp.float32),
        ],
    )
    def sc_pass(xl_hbm, xr_hbm, c_hbm, att_hbm, src_hbm, dst_hbm, out_hbm,
                xlgA, xrgA, cgbA, srcA, dstA, semA,
                xlgB, xrgB, cgbB, srcB, dstB, semB,
                stage, pacc, wbuf, attv, accum):
        cid = lax.axis_index("c")
        sid = lax.axis_index("s")
        tid = sid * NCORE + cid

        pltpu.sync_copy(att_hbm, attv)

        def zero_row(i, carry):
            z = jnp.zeros((16,), jnp.float32)
            for s in range(wrow // 16):
                stage[i, pl.ds(s * 16, 16)] = z
            return carry

        lax.fori_loop(0, CH, zero_row, 0)
        for zb in range(ROWS_PER_TILE // CH):
            pltpu.sync_copy(
                stage, accum.at[pl.ds(sid * ROWS_PER_TILE + zb * CH, CH)])
        plsc.subcore_barrier()

        def load_idx(j, srcv, dstv):
            base = pl.multiple_of((tid * CPT + j) * CH, CH)
            pltpu.sync_copy(src_hbm.at[pl.ds(base, CH)], srcv)
            pltpu.sync_copy(dst_hbm.at[pl.ds(base, CH)], dstv)

        def fire(xlg, xrg, cgb, srcv, dstv, sem):
            pltpu.async_copy(xl_hbm.at[srcv], xlg, sem)
            pltpu.async_copy(xr_hbm.at[dstv], xrg, sem)
            pltpu.async_copy(c_hbm.at[dstv], cgb, sem)

        def drain(xlg, xrg, cgb, srcv, dstv, sem):
            pltpu.make_async_copy(xl_hbm.at[srcv], xlg, sem).wait()
            pltpu.make_async_copy(xr_hbm.at[dstv], xrg, sem).wait()
            pltpu.make_async_copy(c_hbm.at[dstv], cgb, sem).wait()

        def compute(xlg, xrg, cgb, dstv):
            @plsc.parallel_loop(0, CH, unroll=8)
            def e_body(i):
                acc = jnp.zeros((16,), jnp.float32)
                for s in range(nsl32):
                    aw = plsc.bitcast(xlg[i, pl.ds(s * 16, 16)], jnp.bfloat16)
                    bw = plsc.bitcast(xrg[i, pl.ds(s * 16, 16)], jnp.bfloat16)
                    ae, ao = plsc.unpack(
                        aw, format=plsc.PackFormat.INTERLEAVED,
                        preferred_element_type=jnp.float32)
                    be, bo = plsc.unpack(
                        bw, format=plsc.PackFormat.INTERLEAVED,
                        preferred_element_type=jnp.float32)
                    ze = ae + be
                    zo = ao + bo
                    lre = jnp.maximum(ze, 0.2 * ze)
                    lro = jnp.maximum(zo, 0.2 * zo)
                    acc = (acc + lre * attv[pl.ds(s * 32, 16)]
                           + lro * attv[pl.ds(s * 32 + 16, 16)])
                pacc[pl.ds(pl.multiple_of(i * 16, 16), 16)] = acc

            lane = lax.iota(jnp.int32, 16)

            @plsc.parallel_loop(0, CH // 16, unroll=4)
            def w_body(gi):
                off = pl.multiple_of(gi * 16, 16)
                ebase = (lane + off) * 16
                ev = jnp.zeros((16,), jnp.float32)
                for l in range(16):
                    ev = ev + plsc.load_gather(pacc, [ebase + l])
                cg = cgb[pl.ds(off, 16)]
                wbuf[pl.ds(off, 16)] = jnp.exp(ev - cg)

            @plsc.parallel_loop(0, CH, unroll=8)
            def s_body(i):
                idx = jnp.full((16,), i, jnp.int32)
                wv = plsc.load_gather(wbuf, [idx])
                for s in range(nsl32):
                    aw = plsc.bitcast(xlg[i, pl.ds(s * 16, 16)], jnp.bfloat16)
                    ae, ao = plsc.unpack(
                        aw, format=plsc.PackFormat.INTERLEAVED,
                        preferred_element_type=jnp.float32)
                    stage[i, pl.ds(s * 32, 16)] = ae * wv
                    stage[i, pl.ds(s * 32 + 16, 16)] = ao * wv
                stage[i, pl.ds(d, 16)] = wv

            pltpu.sync_copy(stage, accum.at[dstv], add=True)

        A = (xlgA, xrgA, cgbA, srcA, dstA, semA)
        B = (xlgB, xrgB, cgbB, srcB, dstB, semB)

        load_idx(0, srcA, dstA)
        fire(*A)

        def pair_body(t, carry):
            load_idx(2 * t + 1, srcB, dstB)
            fire(*B)
            drain(*A)
            compute(xlgA, xrgA, cgbA, dstA)
            load_idx(2 * t + 2, srcA, dstA)
            fire(*A)
            drain(*B)
            compute(xlgB, xrgB, cgbB, dstB)
            return carry

        lax.fori_loop(0, CPT // 2 - 1, pair_body, 0)
        load_idx(CPT - 1, srcB, dstB)
        fire(*B)
        drain(*A)
        compute(xlgA, xrgA, cgbA, dstA)
        drain(*B)
        compute(xlgB, xrgB, cgbB, dstB)
        plsc.subcore_barrier()

        rows = pl.ds(sid * ROWS_PER_TILE, ROWS_PER_TILE)
        pltpu.sync_copy(accum.at[rows], out_hbm.at[cid].at[rows])

    return sc_pass


_sc_pass_l1 = _make_sc_edge_pass(128)
_sc_pass_l2 = _make_sc_edge_pass(64)


def kernel(x, edge_index, Wl1, Wr1, att1, b1, Wl2, Wr2, att2, b2, Wlin, blin):
    x = x.astype(jnp.float32)
    xp = jnp.pad(x, ((0, NP - N), (0, 0)))
    src = edge_index[0].astype(jnp.int32)
    dst = edge_index[1].astype(jnp.int32)
    npad = EP - E
    pad_dst = (N + (jnp.arange(npad, dtype=jnp.int32) % (NP - N)))
    srcp = jnp.concatenate([src, jnp.zeros((npad,), jnp.int32)])
    dstp = jnp.concatenate([dst, pad_dst])

    xl1, xlp1, xrp1, c1 = _tc_precompute(xp, Wl1, Wr1, att1)
    acc1 = _sc_pass_l1(xlp1, xrp1, c1.reshape(NP), att1, srcp, dstp)
    xl2, xlp2, xrp2, c2 = _tc_combine_next(acc1[0], acc1[1], xl1, b1,
                                           Wl2, Wr2, att2)
    acc2 = _sc_pass_l2(xlp2, xrp2, c2.reshape(NP), att2, srcp, dstp)
    y = _tc_combine_head(acc2[0], acc2[1], xl2, b2, Wlin.reshape(-1), blin)
    return y.reshape(NP)[:N]
